# baseline (speedup 1.0000x reference)
import functools

import jax
import jax.numpy as jnp
from jax import lax
from jax.experimental import pallas as pl
from jax.experimental.pallas import tpu as pltpu
from jax.experimental.pallas import tpu_sc as plsc

B, S, D, M, C = 4, 8192, 2048, 1024, 1000

NC, NS = 2, 16
NW = NC * NS
PB = NW // B
PW = M // PB

BLK = 512
NBLK = B * S // BLK
CPAD = 1008

_mesh = plsc.VectorSubcoreMesh(core_axis_name="c", subcore_axis_name="s")


@functools.partial(
    pl.kernel,
    mesh=_mesh,
    out_type=(
        jax.ShapeDtypeStruct((B * M,), jnp.int32),
        jax.ShapeDtypeStruct((B * M,), jnp.int32),
    ),
    scratch_types=[
        pltpu.VMEM((PW,), jnp.int32),
        pltpu.VMEM((PW,), jnp.int32),
        pltpu.VMEM((PW,), jnp.int32),
        pltpu.SemaphoreType.DMA,
    ],
)
def _sc_prep(cats_hbm, pos_hbm, rows_out, cats_out, pos_v, idx_v, cat_v, sem):
    wid = lax.axis_index("s") * NC + lax.axis_index("c")
    b = wid // PB
    base = wid * PW

    pltpu.sync_copy(pos_hbm.at[pl.ds(base, PW)], pos_v)
    for g in range(PW // 16):
        idx_v[pl.ds(g * 16, 16)] = pos_v[pl.ds(g * 16, 16)] + b * S

    pltpu.async_copy(cats_hbm.at[idx_v], cat_v, sem).wait()

    pltpu.sync_copy(idx_v, rows_out.at[pl.ds(base, PW)])
    pltpu.sync_copy(cat_v, cats_out.at[pl.ds(base, PW)])


def _fused_body(rows_sref, cats_sref, starts_sref,
                in_ref, emb_any, out_ref, emb_vmem, sem):
    i = pl.program_id(0)

    @pl.when(i == 0)
    def _():
        pltpu.async_copy(emb_any, emb_vmem, sem).wait()

    out_ref[...] = in_ref[...]

    def patch(k, carry):
        r = rows_sref[k] - i * BLK
        c = cats_sref[k]
        out_ref[pl.ds(r, 1), :] = emb_vmem[pl.ds(c, 1), :]
        return carry

    lax.fori_loop(starts_sref[i], starts_sref[i + 1], patch, 0)


_fused = pl.pallas_call(
    _fused_body,
    grid_spec=pltpu.PrefetchScalarGridSpec(
        num_scalar_prefetch=3,
        grid=(NBLK,),
        in_specs=[
            pl.BlockSpec((BLK, D), lambda i, *_: (i, 0)),
            pl.BlockSpec(memory_space=pl.ANY),
        ],
        out_specs=pl.BlockSpec((BLK, D), lambda i, *_: (i, 0)),
        scratch_shapes=[
            pltpu.VMEM((CPAD, D), jnp.float32),
            pltpu.SemaphoreType.DMA,
        ],
    ),
    out_shape=jax.ShapeDtypeStruct((B * S, D), jnp.float32),
)


def kernel(inputs_0, categories, mask_positions, tokens_embedding):
    pos = mask_positions[..., 0].reshape(B * M)
    cats = categories.reshape(B * S)
    rows, rcats = _sc_prep(cats, pos)

    key = jnp.sort((rows << 11) | rcats)
    srows = key >> 11
    scats = key & 2047
    starts = jnp.searchsorted(
        srows, jnp.arange(NBLK + 1, dtype=jnp.int32) * BLK).astype(jnp.int32)

    emb = jnp.pad(tokens_embedding, ((0, CPAD - C), (0, 0)))
    out = _fused(srows, scats, starts, inputs_0.reshape(B * S, D), emb)
    return out.reshape(B, S, D)


# speedup vs baseline: 1.0737x; 1.0737x over previous
import functools

import jax
import jax.numpy as jnp
from jax import lax
from jax.experimental import pallas as pl
from jax.experimental.pallas import tpu as pltpu
from jax.experimental.pallas import tpu_sc as plsc

B, S, D, M, C = 4, 8192, 2048, 1024, 1000

NC, NS = 2, 16
NW = NC * NS
PB = NW // B
PW = M // PB

BLK = 512
NBLK = B * S // BLK
CPAD = 1008

_mesh = plsc.VectorSubcoreMesh(core_axis_name="c", subcore_axis_name="s")


@functools.partial(
    pl.kernel,
    mesh=_mesh,
    out_type=(
        jax.ShapeDtypeStruct((B * M,), jnp.int32),
        jax.ShapeDtypeStruct((B * M,), jnp.int32),
    ),
    scratch_types=[
        pltpu.VMEM((PW,), jnp.int32),
        pltpu.VMEM((PW,), jnp.int32),
        pltpu.VMEM((PW,), jnp.int32),
        pltpu.SemaphoreType.DMA,
    ],
)
def _sc_prep(cats_hbm, pos_hbm, rows_out, cats_out, pos_v, idx_v, cat_v, sem):
    wid = lax.axis_index("s") * NC + lax.axis_index("c")
    b = wid // PB
    base = wid * PW

    pltpu.sync_copy(pos_hbm.at[pl.ds(base, PW)], pos_v)
    for g in range(PW // 16):
        idx_v[pl.ds(g * 16, 16)] = pos_v[pl.ds(g * 16, 16)] + b * S

    pltpu.async_copy(cats_hbm.at[idx_v], cat_v, sem).wait()

    pltpu.sync_copy(idx_v, rows_out.at[pl.ds(base, PW)])
    pltpu.sync_copy(cat_v, cats_out.at[pl.ds(base, PW)])


def _fused_body(rows_sref, cats_sref, starts_sref,
                in_ref, emb_any, out_ref, emb_vmem, sem):
    i = pl.program_id(0)

    @pl.when(i == 0)
    def _():
        pltpu.async_copy(emb_any, emb_vmem, sem).wait()

    out_ref[...] = in_ref[...]

    def patch(k, carry):
        r = rows_sref[k] - i * BLK
        c = cats_sref[k]
        out_ref[pl.ds(r, 1), :] = emb_vmem[pl.ds(c, 1), :]
        return carry

    lax.fori_loop(starts_sref[i], starts_sref[i + 1], patch, 0)


_fused = pl.pallas_call(
    _fused_body,
    grid_spec=pltpu.PrefetchScalarGridSpec(
        num_scalar_prefetch=3,
        grid=(NBLK,),
        in_specs=[
            pl.BlockSpec((BLK, D), lambda i, *_: (i, 0)),
            pl.BlockSpec(memory_space=pl.ANY),
        ],
        out_specs=pl.BlockSpec((BLK, D), lambda i, *_: (i, 0)),
        scratch_shapes=[
            pltpu.VMEM((C, D), jnp.float32),
            pltpu.SemaphoreType.DMA,
        ],
    ),
    out_shape=jax.ShapeDtypeStruct((B * S, D), jnp.float32),
)


def kernel(inputs_0, categories, mask_positions, tokens_embedding):
    pos = mask_positions[..., 0].reshape(B * M)
    cats = categories.reshape(B * S)
    rows, rcats = _sc_prep(cats, pos)

    key = jnp.sort((rows << 11) | rcats)
    srows = key >> 11
    scats = key & 2047
    blk_of = key >> 20
    counts = jnp.sum(
        blk_of[:, None] == jnp.arange(NBLK, dtype=jnp.int32)[None, :],
        axis=0, dtype=jnp.int32)
    starts = jnp.concatenate(
        [jnp.zeros((1,), jnp.int32), jnp.cumsum(counts, dtype=jnp.int32)])

    out = _fused(srows, scats, starts, inputs_0.reshape(B * S, D),
                 tokens_embedding)
    return out.reshape(B, S, D)


# speedup vs baseline: 1.0743x; 1.0006x over previous
import functools

import jax
import jax.numpy as jnp
from jax import lax
from jax.experimental import pallas as pl
from jax.experimental.pallas import tpu as pltpu
from jax.experimental.pallas import tpu_sc as plsc

B, S, D, M, C = 4, 8192, 2048, 1024, 1000

NC, NS = 2, 16
NW = NC * NS
PB = NW // B
PW = M // PB

BLK = 512
NBLK = B * S // BLK
CPAD = 1008

_mesh = plsc.VectorSubcoreMesh(core_axis_name="c", subcore_axis_name="s")


@functools.partial(
    pl.kernel,
    mesh=_mesh,
    out_type=(
        jax.ShapeDtypeStruct((B * M,), jnp.int32),
        jax.ShapeDtypeStruct((B * M,), jnp.int32),
    ),
    scratch_types=[
        pltpu.VMEM((PW,), jnp.int32),
        pltpu.VMEM((PW,), jnp.int32),
        pltpu.VMEM((PW,), jnp.int32),
        pltpu.SemaphoreType.DMA,
    ],
)
def _sc_prep(cats_hbm, pos_hbm, rows_out, cats_out, pos_v, idx_v, cat_v, sem):
    wid = lax.axis_index("s") * NC + lax.axis_index("c")
    b = wid // PB
    base = wid * PW

    pltpu.sync_copy(pos_hbm.at[pl.ds(base, PW)], pos_v)
    for g in range(PW // 16):
        idx_v[pl.ds(g * 16, 16)] = pos_v[pl.ds(g * 16, 16)] + b * S

    pltpu.async_copy(cats_hbm.at[idx_v], cat_v, sem).wait()

    pltpu.sync_copy(idx_v, rows_out.at[pl.ds(base, PW)])
    pltpu.sync_copy(cat_v, cats_out.at[pl.ds(base, PW)])


def _fused_body(rows_sref, cats_sref, starts_sref,
                in_ref, emb_any, out_ref, emb_vmem, sem):
    i = pl.program_id(0)

    @pl.when(i == 0)
    def _():
        pltpu.async_copy(emb_any, emb_vmem, sem).wait()

    out_ref[...] = in_ref[...]

    def patch(k, carry):
        r = rows_sref[k] - i * BLK
        c = cats_sref[k]
        out_ref[pl.ds(r, 1), :] = emb_vmem[pl.ds(c, 1), :]
        return carry

    lax.fori_loop(starts_sref[i], starts_sref[i + 1], patch, 0)


_fused = pl.pallas_call(
    _fused_body,
    grid_spec=pltpu.PrefetchScalarGridSpec(
        num_scalar_prefetch=3,
        grid=(NBLK,),
        in_specs=[
            pl.BlockSpec((BLK, D), lambda i, *_: (i, 0)),
            pl.BlockSpec(memory_space=pl.ANY),
        ],
        out_specs=pl.BlockSpec((BLK, D), lambda i, *_: (i, 0)),
        scratch_shapes=[
            pltpu.VMEM((C, D), jnp.float32),
            pltpu.SemaphoreType.DMA,
        ],
    ),
    out_shape=jax.ShapeDtypeStruct((B * S, D), jnp.float32),
)


def kernel(inputs_0, categories, mask_positions, tokens_embedding):
    pos = mask_positions[..., 0].reshape(B * M)
    cats = categories.reshape(B * S)
    rows, rcats = _sc_prep(cats, pos)

    key = jnp.sort((rows << 11) | rcats)
    srows = key >> 11
    scats = key & 2047
    blk_of = key >> 20
    counts = jnp.sum(
        jnp.arange(NBLK, dtype=jnp.int32)[:, None] == blk_of[None, :],
        axis=1, dtype=jnp.int32)
    starts = jnp.concatenate(
        [jnp.zeros((1,), jnp.int32), jnp.cumsum(counts, dtype=jnp.int32)])

    out = _fused(srows, scats, starts, inputs_0.reshape(B * S, D),
                 tokens_embedding)
    return out.reshape(B, S, D)


# speedup vs baseline: 1.0869x; 1.0118x over previous
import functools

import jax
import jax.numpy as jnp
from jax import lax
from jax.experimental import pallas as pl
from jax.experimental.pallas import tpu as pltpu
from jax.experimental.pallas import tpu_sc as plsc

B, S, D, M, C = 4, 8192, 2048, 1024, 1000

NC, NS = 2, 16
NW = NC * NS
PB = NW // B
PW = M // PB

BLK = 1024
NBLK = B * S // BLK
CPAD = 1008

_mesh = plsc.VectorSubcoreMesh(core_axis_name="c", subcore_axis_name="s")


@functools.partial(
    pl.kernel,
    mesh=_mesh,
    out_type=(
        jax.ShapeDtypeStruct((B * M,), jnp.int32),
        jax.ShapeDtypeStruct((B * M,), jnp.int32),
    ),
    scratch_types=[
        pltpu.VMEM((PW,), jnp.int32),
        pltpu.VMEM((PW,), jnp.int32),
        pltpu.VMEM((PW,), jnp.int32),
        pltpu.SemaphoreType.DMA,
    ],
)
def _sc_prep(cats_hbm, pos_hbm, rows_out, cats_out, pos_v, idx_v, cat_v, sem):
    wid = lax.axis_index("s") * NC + lax.axis_index("c")
    b = wid // PB
    base = wid * PW

    pltpu.sync_copy(pos_hbm.at[pl.ds(base, PW)], pos_v)
    for g in range(PW // 16):
        idx_v[pl.ds(g * 16, 16)] = pos_v[pl.ds(g * 16, 16)] + b * S

    pltpu.async_copy(cats_hbm.at[idx_v], cat_v, sem).wait()

    pltpu.sync_copy(idx_v, rows_out.at[pl.ds(base, PW)])
    pltpu.sync_copy(cat_v, cats_out.at[pl.ds(base, PW)])


def _fused_body(rows_sref, cats_sref, starts_sref,
                in_ref, emb_any, out_ref, emb_vmem, sem):
    i = pl.program_id(0)

    @pl.when(i == 0)
    def _():
        pltpu.async_copy(emb_any, emb_vmem, sem).wait()

    out_ref[...] = in_ref[...]

    def patch(k, carry):
        r = rows_sref[k] - i * BLK
        c = cats_sref[k]
        out_ref[pl.ds(r, 1), :] = emb_vmem[pl.ds(c, 1), :]
        return carry

    lax.fori_loop(starts_sref[i], starts_sref[i + 1], patch, 0)


_fused = pl.pallas_call(
    _fused_body,
    grid_spec=pltpu.PrefetchScalarGridSpec(
        num_scalar_prefetch=3,
        grid=(NBLK,),
        in_specs=[
            pl.BlockSpec((BLK, D), lambda i, *_: (i, 0)),
            pl.BlockSpec(memory_space=pl.ANY),
        ],
        out_specs=pl.BlockSpec((BLK, D), lambda i, *_: (i, 0)),
        scratch_shapes=[
            pltpu.VMEM((C, D), jnp.float32),
            pltpu.SemaphoreType.DMA,
        ],
    ),
    out_shape=jax.ShapeDtypeStruct((B * S, D), jnp.float32),
)


def kernel(inputs_0, categories, mask_positions, tokens_embedding):
    pos = mask_positions[..., 0].reshape(B * M)
    cats = categories.reshape(B * S)
    rows, rcats = _sc_prep(cats, pos)

    key = jnp.sort((rows << 11) | rcats)
    srows = key >> 11
    scats = key & 2047
    blk_of = key >> 21
    counts = jnp.sum(
        jnp.arange(NBLK, dtype=jnp.int32)[:, None] == blk_of[None, :],
        axis=1, dtype=jnp.int32)
    starts = jnp.concatenate(
        [jnp.zeros((1,), jnp.int32), jnp.cumsum(counts, dtype=jnp.int32)])

    out = _fused(srows, scats, starts, inputs_0.reshape(B * S, D),
                 tokens_embedding)
    return out.reshape(B, S, D)
